# Initial kernel scaffold; baseline (speedup 1.0000x reference)
import jax
import jax.numpy as jnp
from jax.experimental import pallas as pl


def kernel(x, edge_index):
    raise NotImplementedError("write your pallas kernel here")


# speedup vs baseline: 4.3427x; 4.3427x over previous
import functools

import jax
import jax.numpy as jnp
from jax import lax
from jax.experimental import pallas as pl
from jax.experimental.pallas import tpu as pltpu
from jax.experimental.pallas import tpu_sc as plsc

_N = 10000
_D = 128
_E = 320000
_NC = 2
_NS = 16
_NW = _NC * _NS
_EPW = _E // _NW
_CH = 80
_NCHUNK = _EPW // _CH
_STRIPE = 624
_ZR = 16


def _sc_partial_sums(x, idx3):
    mesh = plsc.VectorSubcoreMesh(core_axis_name="c", subcore_axis_name="s")

    @functools.partial(
        pl.kernel,
        mesh=mesh,
        out_type=jax.ShapeDtypeStruct((_NC, _N, _D), jnp.float32),
        scratch_types=[
            pltpu.VMEM_SHARED((_N, _D), jnp.float32),
            pltpu.VMEM((_NCHUNK, _CH), jnp.int32),
            pltpu.VMEM((_CH, _D), jnp.float32),
            pltpu.VMEM((_ZR, _D), jnp.float32),
        ],
    )
    def k(x_hbm, idx_hbm, out_hbm, acc_sh, idx_v, xb, zb):
        c = lax.axis_index("c")
        s = lax.axis_index("s")
        wid = c * _NS + s

        pltpu.sync_copy(idx_hbm.at[wid], idx_v)

        for i in range(_ZR):
            for k0 in range(_D // 16):
                zb[i, pl.ds(k0 * 16, 16)] = jnp.zeros((16,), jnp.float32)

        row0 = s * _STRIPE
        nz = jnp.where(s == _NS - 1, (_N - 15 * _STRIPE) // _ZR, _STRIPE // _ZR)

        def zbody(r, carry):
            pltpu.sync_copy(zb, acc_sh.at[pl.ds(row0 + r * _ZR, _ZR)])
            return carry

        lax.fori_loop(0, nz, zbody, 0)
        plsc.subcore_barrier()

        base = wid * _EPW

        def body(j, carry):
            pltpu.sync_copy(x_hbm.at[pl.ds(base + j * _CH, _CH)], xb)
            pltpu.sync_copy(xb, acc_sh.at[idx_v.at[j]], add=True)
            return carry

        lax.fori_loop(0, _NCHUNK, body, 0)
        plsc.subcore_barrier()

        @pl.when(s < _NS - 1)
        def _():
            pltpu.sync_copy(acc_sh.at[pl.ds(row0, _STRIPE)],
                            out_hbm.at[c].at[pl.ds(row0, _STRIPE)])

        @pl.when(s == _NS - 1)
        def _():
            last0 = 15 * _STRIPE
            pltpu.sync_copy(acc_sh.at[pl.ds(last0, _N - 15 * _STRIPE)],
                            out_hbm.at[c].at[pl.ds(last0, _N - 15 * _STRIPE)])

    return k(x, idx3)


def _combine(p):
    def add_body(a_ref, b_ref, o_ref):
        o_ref[...] = a_ref[...] + b_ref[...]

    return pl.pallas_call(
        add_body,
        grid=(10,),
        in_specs=[pl.BlockSpec((_N // 10, _D), lambda i: (i, 0)),
                  pl.BlockSpec((_N // 10, _D), lambda i: (i, 0))],
        out_specs=pl.BlockSpec((_N // 10, _D), lambda i: (i, 0)),
        out_shape=jax.ShapeDtypeStruct((_N, _D), jnp.float32),
    )(p[0], p[1])


def kernel(x, edge_index):
    idx3 = edge_index[0].reshape(_NW, _NCHUNK, _CH)
    p = _sc_partial_sums(x, idx3)
    return _combine(p)
